# baseline (speedup 1.0000x reference)
import functools

import jax
import jax.numpy as jnp
from jax.experimental import pallas as pl
from jax.experimental.pallas import tpu as pltpu

N_BOXES = 20000
PRE_NMS = 12000
POST_NMS = 2000
THRESH = 0.7

B = 512
NB = (PRE_NMS + B - 1) // B
NPAD = NB * B
KCAP = 2048
CHUNK = 64


def _nms_kernel(boxes_ref, scores_ref, im_ref, blob_ref, sco_ref, kept_ref):
    kept_ref[...] = jnp.zeros((KCAP, 8), jnp.float32)

    w = im_ref[0, 1]
    h = im_ref[0, 0]
    row4 = jax.lax.broadcasted_iota(jnp.int32, (4, 1), 0)
    hi_bound = jnp.where(row4 % 2 == 0, w - 1.0, h - 1.0)

    lane = jax.lax.broadcasted_iota(jnp.int32, (1, B), 1)
    row64 = jax.lax.broadcasted_iota(jnp.int32, (CHUNK, 1), 0)
    lane8 = jax.lax.broadcasted_iota(jnp.int32, (1, 8), 1)

    def outer_body(state):
        b, count = state
        blk = boxes_ref[b]
        blk = jnp.minimum(jnp.maximum(blk, 0.0), hi_bound)
        bx1 = blk[0:1, :]
        by1 = blk[1:2, :]
        bx2 = blk[2:3, :]
        by2 = blk[3:4, :]
        barea = (bx2 - bx1 + 1.0) * (by2 - by1 + 1.0)
        bsc = scores_ref[b]
        stacked = jnp.concatenate(
            [blk, bsc, barea, jnp.zeros((2, B), jnp.float32)], axis=0)

        valid0 = ((b * B + lane) < PRE_NMS).astype(jnp.float32)

        nchunks = (count + CHUNK - 1) // CHUNK

        def chunk_body(c, bval):
            kc = kept_ref[pl.ds(c * CHUNK, CHUNK), :]
            kx1 = kc[:, 1:2]
            ky1 = kc[:, 2:3]
            kx2 = kc[:, 3:4]
            ky2 = kc[:, 4:5]
            karea = (kx2 - kx1 + 1.0) * (ky2 - ky1 + 1.0)
            xx1 = jnp.maximum(kx1, bx1)
            yy1 = jnp.maximum(ky1, by1)
            xx2 = jnp.minimum(kx2, bx2)
            yy2 = jnp.minimum(ky2, by2)
            iw = jnp.maximum(0.0, xx2 - xx1 + 1.0)
            ih = jnp.maximum(0.0, yy2 - yy1 + 1.0)
            inter = iw * ih
            iou = inter / (karea + barea - inter)
            rowok = (c * CHUNK + row64) < count
            supp = jnp.where((iou >= THRESH) & rowok, 1.0, 0.0)
            supp = jnp.max(supp, axis=0, keepdims=True)
            return jnp.where(supp > 0.0, 0.0, bval)

        bvalid = jax.lax.fori_loop(0, nchunks, chunk_body, valid0)

        BIG = jnp.int32(2 * B)
        nxt0 = jnp.min(jnp.where(bvalid > 0.0, lane, BIG))

        def inner_cond(st):
            cnt, _, nxt = st
            return (cnt < POST_NMS) & (nxt < BIG)

        def inner_body(st):
            cnt, bval, nxt = st
            one8 = jnp.broadcast_to(lane, (8, B)) == nxt
            vals = jnp.sum(jnp.where(one8, stacked, 0.0), axis=1,
                           keepdims=True)
            x1k = vals[0:1, :]
            y1k = vals[1:2, :]
            x2k = vals[2:3, :]
            y2k = vals[3:4, :]
            sck = vals[4:5, :]
            areak = vals[5:6, :]
            xx1 = jnp.maximum(x1k, bx1)
            yy1 = jnp.maximum(y1k, by1)
            xx2 = jnp.minimum(x2k, bx2)
            yy2 = jnp.minimum(y2k, by2)
            iw = jnp.maximum(0.0, xx2 - xx1 + 1.0)
            ih = jnp.maximum(0.0, yy2 - yy1 + 1.0)
            inter = iw * ih
            iou = inter / (areak + barea - inter)
            bval = jnp.where(iou >= THRESH, 0.0, bval)
            nxt2 = jnp.min(jnp.where(bval > 0.0, lane, BIG))
            rowv = (jnp.where(lane8 == 1, x1k, 0.0)
                    + jnp.where(lane8 == 2, y1k, 0.0)
                    + jnp.where(lane8 == 3, x2k, 0.0)
                    + jnp.where(lane8 == 4, y2k, 0.0)
                    + jnp.where(lane8 == 5, sck, 0.0))
            kept_ref[pl.ds(cnt, 1), :] = rowv
            return cnt + 1, bval, nxt2

        count, _, _ = jax.lax.while_loop(
            inner_cond, inner_body, (count, bvalid, nxt0))
        return b + 1, count

    def outer_cond(state):
        b, count = state
        return (b < NB) & (count < POST_NMS)

    jax.lax.while_loop(outer_cond, outer_body, (jnp.int32(0), jnp.int32(0)))

    blob_ref[...] = kept_ref[0:POST_NMS, 0:5]
    sco_ref[...] = kept_ref[0:POST_NMS, 5:6]


@functools.partial(jax.jit, static_argnames=())
def kernel(boxes, scores, im_info):
    scores_sorted, order = jax.lax.top_k(scores, PRE_NMS)
    props = boxes[order]
    boxes_t = jnp.zeros((4, NPAD), jnp.float32).at[:, :PRE_NMS].set(props.T)
    boxes_t3 = boxes_t.reshape(4, NB, B).transpose(1, 0, 2)
    sc_p = jnp.zeros((1, NPAD), jnp.float32).at[:, :PRE_NMS].set(
        scores_sorted[None, :])
    sc_p3 = sc_p.reshape(1, NB, B).transpose(1, 0, 2)

    blob, out_scores = pl.pallas_call(
        _nms_kernel,
        out_shape=[
            jax.ShapeDtypeStruct((POST_NMS, 5), jnp.float32),
            jax.ShapeDtypeStruct((POST_NMS, 1), jnp.float32),
        ],
        scratch_shapes=[pltpu.VMEM((KCAP, 8), jnp.float32)],
    )(boxes_t3, sc_p3, im_info.reshape(1, 3))
    return blob, out_scores


# speedup vs baseline: 1.1807x; 1.1807x over previous
import functools

import jax
import jax.numpy as jnp
from jax.experimental import pallas as pl
from jax.experimental.pallas import tpu as pltpu

N_BOXES = 20000
PRE_NMS = 12000
POST_NMS = 2000
THRESH = 0.7

B = 128
NB = (PRE_NMS + B - 1) // B
NPAD = NB * B
KCAP = 2048
CHUNK = 64


def _nms_kernel(lanes_ref, cols_ref, im_ref, blob_ref, sco_ref,
                kept_ref, m_ref, col_ref):
    kept_ref[...] = jnp.zeros((KCAP, 8), jnp.float32)

    w = im_ref[0, 1]
    h = im_ref[0, 0]
    row4 = jax.lax.broadcasted_iota(jnp.int32, (4, 1), 0)
    hi_bound = jnp.where(row4 % 2 == 0, w - 1.0, h - 1.0)
    col8 = jax.lax.broadcasted_iota(jnp.int32, (1, 8), 1)
    colhi = jnp.where(col8 % 2 == 0, w - 1.0, h - 1.0)

    lane = jax.lax.broadcasted_iota(jnp.int32, (1, B), 1)
    row64 = jax.lax.broadcasted_iota(jnp.int32, (CHUNK, 1), 0)

    def outer_body(state):
        b, count = state
        blk = lanes_ref[b]
        blk = jnp.minimum(jnp.maximum(blk, 0.0), hi_bound)
        bx1 = blk[0:1, :]
        by1 = blk[1:2, :]
        bx2 = blk[2:3, :]
        by2 = blk[3:4, :]
        barea = (bx2 - bx1 + 1.0) * (by2 - by1 + 1.0)

        cb = cols_ref[b]
        cb = jnp.where(col8 < 4,
                       jnp.minimum(jnp.maximum(cb, 0.0), colhi), cb)
        col_ref[...] = cb
        cx1 = cb[:, 0:1]
        cy1 = cb[:, 1:2]
        cx2 = cb[:, 2:3]
        cy2 = cb[:, 3:4]
        carea = (cx2 - cx1 + 1.0) * (cy2 - cy1 + 1.0)

        mxx1 = jnp.maximum(cx1, bx1)
        myy1 = jnp.maximum(cy1, by1)
        mxx2 = jnp.minimum(cx2, bx2)
        myy2 = jnp.minimum(cy2, by2)
        miw = jnp.maximum(0.0, mxx2 - mxx1 + 1.0)
        mih = jnp.maximum(0.0, myy2 - myy1 + 1.0)
        minter = miw * mih
        m_ref[...] = minter / (carea + barea - minter)

        valid0 = ((b * B + lane) < PRE_NMS).astype(jnp.float32)

        nchunks = (count + CHUNK - 1) // CHUNK

        def chunk_body(c, bval):
            kc = kept_ref[pl.ds(c * CHUNK, CHUNK), :]
            kx1 = kc[:, 1:2]
            ky1 = kc[:, 2:3]
            kx2 = kc[:, 3:4]
            ky2 = kc[:, 4:5]
            karea = (kx2 - kx1 + 1.0) * (ky2 - ky1 + 1.0)
            xx1 = jnp.maximum(kx1, bx1)
            yy1 = jnp.maximum(ky1, by1)
            xx2 = jnp.minimum(kx2, bx2)
            yy2 = jnp.minimum(ky2, by2)
            iw = jnp.maximum(0.0, xx2 - xx1 + 1.0)
            ih = jnp.maximum(0.0, yy2 - yy1 + 1.0)
            inter = iw * ih
            iou = inter / (karea + barea - inter)
            rowok = (c * CHUNK + row64) < count
            supp = jnp.where((iou >= THRESH) & rowok, 1.0, 0.0)
            supp = jnp.max(supp, axis=0, keepdims=True)
            return jnp.where(supp > 0.0, 0.0, bval)

        bvalid = jax.lax.fori_loop(0, nchunks, chunk_body, valid0)

        BIG = jnp.int32(2 * B)
        nxt0 = jnp.min(jnp.where(bvalid > 0.0, lane, BIG))

        def inner_cond(st):
            cnt, _, nxt = st
            return (cnt < POST_NMS) & (nxt < BIG)

        def inner_body(st):
            cnt, bval, nxt = st
            iou_row = m_ref[pl.ds(nxt, 1), :]
            bval = jnp.where(iou_row >= THRESH, 0.0, bval)
            nxt2 = jnp.min(jnp.where(bval > 0.0, lane, BIG))
            krow = col_ref[pl.ds(nxt, 1), :]
            rowv = jnp.concatenate([krow[:, 7:8], krow[:, 0:7]], axis=1)
            kept_ref[pl.ds(cnt, 1), :] = rowv
            return cnt + 1, bval, nxt2

        count, _, _ = jax.lax.while_loop(
            inner_cond, inner_body, (count, bvalid, nxt0))
        return b + 1, count

    def outer_cond(state):
        b, count = state
        return (b < NB) & (count < POST_NMS)

    jax.lax.while_loop(outer_cond, outer_body, (jnp.int32(0), jnp.int32(0)))

    blob_ref[...] = kept_ref[0:POST_NMS, 0:5]
    sco_ref[...] = kept_ref[0:POST_NMS, 5:6]


@functools.partial(jax.jit, static_argnames=())
def kernel(boxes, scores, im_info):
    scores_sorted, order = jax.lax.top_k(scores, PRE_NMS)
    props = boxes[order]
    lanes = jnp.zeros((4, NPAD), jnp.float32).at[:, :PRE_NMS].set(props.T)
    lanes3 = lanes.reshape(4, NB, B).transpose(1, 0, 2)
    cols = jnp.zeros((NPAD, 8), jnp.float32)
    cols = cols.at[:PRE_NMS, 0:4].set(props)
    cols = cols.at[:PRE_NMS, 4].set(scores_sorted)
    cols3 = cols.reshape(NB, B, 8)

    blob, out_scores = pl.pallas_call(
        _nms_kernel,
        out_shape=[
            jax.ShapeDtypeStruct((POST_NMS, 5), jnp.float32),
            jax.ShapeDtypeStruct((POST_NMS, 1), jnp.float32),
        ],
        scratch_shapes=[
            pltpu.VMEM((KCAP, 8), jnp.float32),
            pltpu.VMEM((B, B), jnp.float32),
            pltpu.VMEM((B, 8), jnp.float32),
        ],
    )(lanes3, cols3, im_info.reshape(1, 3))
    return blob, out_scores


# speedup vs baseline: 3.4930x; 2.9586x over previous
import functools

import jax
import jax.numpy as jnp
from jax.experimental import pallas as pl
from jax.experimental.pallas import tpu as pltpu

N_BOXES = 20000
PRE_NMS = 12000
POST_NMS = 2000
THRESH = 0.7

B = 128
NB = (PRE_NMS + B - 1) // B
NPAD = NB * B
KCAP = 2176
CHUNK = 64

_HI = jax.lax.Precision.HIGHEST


def _nms_kernel(lanes_ref, cols_ref, im_ref, blob_ref, sco_ref,
                kept_ref, m_ref):
    kept_ref[...] = jnp.zeros((KCAP, 8), jnp.float32)

    w = im_ref[0, 1]
    h = im_ref[0, 0]
    row4 = jax.lax.broadcasted_iota(jnp.int32, (4, 1), 0)
    hi_bound = jnp.where(row4 % 2 == 0, w - 1.0, h - 1.0)
    col8 = jax.lax.broadcasted_iota(jnp.int32, (1, 8), 1)
    colhi = jnp.where(col8 % 2 == 0, w - 1.0, h - 1.0)

    lane = jax.lax.broadcasted_iota(jnp.int32, (1, B), 1)
    row64 = jax.lax.broadcasted_iota(jnp.int32, (CHUNK, 1), 0)
    subi = jax.lax.broadcasted_iota(jnp.int32, (B, B), 0)
    lani = jax.lax.broadcasted_iota(jnp.int32, (B, B), 1)
    trif = jnp.where(subi < lani, 1.0, 0.0).astype(jnp.float32)
    subf = subi.astype(jnp.float32)

    def outer_body(state):
        b, count = state
        blk = lanes_ref[b]
        blk = jnp.minimum(jnp.maximum(blk, 0.0), hi_bound)
        bx1 = blk[0:1, :]
        by1 = blk[1:2, :]
        bx2 = blk[2:3, :]
        by2 = blk[3:4, :]
        barea = (bx2 - bx1 + 1.0) * (by2 - by1 + 1.0)

        cb = cols_ref[b]
        cb = jnp.where(col8 < 4,
                       jnp.minimum(jnp.maximum(cb, 0.0), colhi), cb)
        cx1 = cb[:, 0:1]
        cy1 = cb[:, 1:2]
        cx2 = cb[:, 2:3]
        cy2 = cb[:, 3:4]
        carea = (cx2 - cx1 + 1.0) * (cy2 - cy1 + 1.0)

        mxx1 = jnp.maximum(cx1, bx1)
        myy1 = jnp.maximum(cy1, by1)
        mxx2 = jnp.minimum(cx2, bx2)
        myy2 = jnp.minimum(cy2, by2)
        miw = jnp.maximum(0.0, mxx2 - mxx1 + 1.0)
        mih = jnp.maximum(0.0, myy2 - myy1 + 1.0)
        minter = miw * mih
        miou = minter / (carea + barea - minter)
        m_ref[...] = jnp.where((miou >= THRESH) & (subi < lani), 1.0, 0.0)

        valid0 = ((b * B + lane) < PRE_NMS).astype(jnp.float32)

        nchunks = (count + CHUNK - 1) // CHUNK

        def chunk_body(c, bval):
            kc = kept_ref[pl.ds(c * CHUNK, CHUNK), :]
            kx1 = kc[:, 1:2]
            ky1 = kc[:, 2:3]
            kx2 = kc[:, 3:4]
            ky2 = kc[:, 4:5]
            karea = (kx2 - kx1 + 1.0) * (ky2 - ky1 + 1.0)
            xx1 = jnp.maximum(kx1, bx1)
            yy1 = jnp.maximum(ky1, by1)
            xx2 = jnp.minimum(kx2, bx2)
            yy2 = jnp.minimum(ky2, by2)
            iw = jnp.maximum(0.0, xx2 - xx1 + 1.0)
            ih = jnp.maximum(0.0, yy2 - yy1 + 1.0)
            inter = iw * ih
            iou = inter / (karea + barea - inter)
            rowok = (c * CHUNK + row64) < count
            supp = jnp.where((iou >= THRESH) & rowok, 1.0, 0.0)
            supp = jnp.max(supp, axis=0, keepdims=True)
            return jnp.where(supp > 0.0, 0.0, bval)

        bvalid = jax.lax.fori_loop(0, nchunks, chunk_body, valid0)

        def rcond(st):
            kcand, _ = st
            return jnp.max(kcand) > 0.0

        def rbody(st):
            kcand, kf = st
            tm = m_ref[...]
            pred = jnp.dot(kcand, tm, precision=_HI)
            dfin = jnp.where(pred > 0.0, 0.0, kcand)
            rsup = jnp.dot(dfin, tm, precision=_HI)
            kcand = jnp.where((dfin + rsup) > 0.0, 0.0, kcand)
            return kcand, kf + dfin

        _, kf = jax.lax.while_loop(
            rcond, rbody, (bvalid, jnp.zeros((1, B), jnp.float32)))

        posf = jnp.dot(kf, trif, precision=_HI)
        scat = jnp.where((jnp.broadcast_to(posf, (B, B)) == subf)
                         & (jnp.broadcast_to(kf, (B, B)) > 0.0), 1.0, 0.0)
        shifted = jnp.concatenate(
            [jnp.zeros((B, 1), jnp.float32), cb[:, 0:5],
             jnp.zeros((B, 2), jnp.float32)], axis=1)
        compacted = jnp.dot(scat, shifted, precision=_HI)
        kept_ref[pl.ds(count, B), :] = compacted
        nk = jnp.sum(kf).astype(jnp.int32)
        count = jnp.minimum(count + nk, POST_NMS)
        return b + 1, count

    def outer_cond(state):
        b, count = state
        return (b < NB) & (count < POST_NMS)

    jax.lax.while_loop(outer_cond, outer_body, (jnp.int32(0), jnp.int32(0)))

    blob_ref[...] = kept_ref[0:POST_NMS, 0:5]
    sco_ref[...] = kept_ref[0:POST_NMS, 5:6]


@functools.partial(jax.jit, static_argnames=())
def kernel(boxes, scores, im_info):
    scores_sorted, order = jax.lax.top_k(scores, PRE_NMS)
    props = boxes[order]
    lanes = jnp.zeros((4, NPAD), jnp.float32).at[:, :PRE_NMS].set(props.T)
    lanes3 = lanes.reshape(4, NB, B).transpose(1, 0, 2)
    cols = jnp.zeros((NPAD, 8), jnp.float32)
    cols = cols.at[:PRE_NMS, 0:4].set(props)
    cols = cols.at[:PRE_NMS, 4].set(scores_sorted)
    cols3 = cols.reshape(NB, B, 8)

    blob, out_scores = pl.pallas_call(
        _nms_kernel,
        out_shape=[
            jax.ShapeDtypeStruct((POST_NMS, 5), jnp.float32),
            jax.ShapeDtypeStruct((POST_NMS, 1), jnp.float32),
        ],
        scratch_shapes=[
            pltpu.VMEM((KCAP, 8), jnp.float32),
            pltpu.VMEM((B, B), jnp.float32),
        ],
    )(lanes3, cols3, im_info.reshape(1, 3))
    return blob, out_scores


# speedup vs baseline: 4.6968x; 1.3446x over previous
import functools

import jax
import jax.numpy as jnp
from jax.experimental import pallas as pl
from jax.experimental.pallas import tpu as pltpu

N_BOXES = 20000
PRE_NMS = 12000
POST_NMS = 2000
THRESH = 0.7

B = 128
NB = (PRE_NMS + B - 1) // B
NPAD = NB * B
KCAP = 2176
CHUNK = 128

_HI = jax.lax.Precision.HIGHEST


def _nms_kernel(lanes_ref, cols_ref, im_ref, blob_ref, sco_ref, kept_ref):
    kept_ref[...] = jnp.zeros((KCAP, 8), jnp.float32)

    w = im_ref[0, 1]
    h = im_ref[0, 0]
    row4 = jax.lax.broadcasted_iota(jnp.int32, (4, 1), 0)
    hi_bound = jnp.where(row4 % 2 == 0, w - 1.0, h - 1.0)
    col8 = jax.lax.broadcasted_iota(jnp.int32, (1, 8), 1)
    colhi = jnp.where(col8 % 2 == 0, w - 1.0, h - 1.0)

    lane = jax.lax.broadcasted_iota(jnp.int32, (1, B), 1)
    row64 = jax.lax.broadcasted_iota(jnp.int32, (CHUNK, 1), 0)
    subi = jax.lax.broadcasted_iota(jnp.int32, (B, B), 0)
    lani = jax.lax.broadcasted_iota(jnp.int32, (B, B), 1)
    trif = jnp.where(subi < lani, 1.0, 0.0).astype(jnp.float32)
    subf = subi.astype(jnp.float32)

    def outer_body(state):
        b, count = state
        blk = lanes_ref[b]
        blk = jnp.minimum(jnp.maximum(blk, 0.0), hi_bound)
        bx1 = blk[0:1, :]
        by1 = blk[1:2, :]
        bx2 = blk[2:3, :]
        by2 = blk[3:4, :]
        barea = (bx2 - bx1 + 1.0) * (by2 - by1 + 1.0)

        cb = cols_ref[b]
        cb = jnp.where(col8 < 4,
                       jnp.minimum(jnp.maximum(cb, 0.0), colhi), cb)
        cx1 = cb[:, 0:1]
        cy1 = cb[:, 1:2]
        cx2 = cb[:, 2:3]
        cy2 = cb[:, 3:4]
        carea = (cx2 - cx1 + 1.0) * (cy2 - cy1 + 1.0)

        mxx1 = jnp.maximum(cx1, bx1)
        myy1 = jnp.maximum(cy1, by1)
        mxx2 = jnp.minimum(cx2, bx2)
        myy2 = jnp.minimum(cy2, by2)
        miw = jnp.maximum(0.0, mxx2 - mxx1 + 1.0)
        mih = jnp.maximum(0.0, myy2 - myy1 + 1.0)
        minter = miw * mih
        miou = minter / (carea + barea - minter)
        tm = jnp.where((miou >= THRESH) & (subi < lani), 1.0, 0.0)

        valid0 = ((b * B + lane) < PRE_NMS).astype(jnp.float32)

        nchunks = (count + CHUNK - 1) // CHUNK

        def chunk_body(c, bval):
            kc = kept_ref[pl.ds(c * CHUNK, CHUNK), :]
            kx1 = kc[:, 1:2]
            ky1 = kc[:, 2:3]
            kx2 = kc[:, 3:4]
            ky2 = kc[:, 4:5]
            karea = kc[:, 6:7]
            xx1 = jnp.maximum(kx1, bx1)
            yy1 = jnp.maximum(ky1, by1)
            xx2 = jnp.minimum(kx2, bx2)
            yy2 = jnp.minimum(ky2, by2)
            iw = jnp.maximum(0.0, xx2 - xx1 + 1.0)
            ih = jnp.maximum(0.0, yy2 - yy1 + 1.0)
            inter = iw * ih
            iou = inter / (karea + barea - inter)
            rowok = (c * CHUNK + row64) < count
            supp = jnp.where((iou >= THRESH) & rowok, 1.0, 0.0)
            supp = jnp.max(supp, axis=0, keepdims=True)
            return jnp.where(supp > 0.0, 0.0, bval)

        bvalid = jax.lax.fori_loop(0, nchunks, chunk_body, valid0)

        def rcond(st):
            kcand, _ = st
            return jnp.max(kcand) > 0.0

        def rbody(st):
            kcand, kf = st
            pred = jnp.dot(kcand, tm)
            dfin = jnp.where(pred > 0.0, 0.0, kcand)
            rsup = jnp.dot(dfin, tm)
            kcand = jnp.where((dfin + rsup) > 0.0, 0.0, kcand)
            return kcand, kf + dfin

        _, kf = jax.lax.while_loop(
            rcond, rbody, (bvalid, jnp.zeros((1, B), jnp.float32)))

        posf = jnp.dot(kf, trif)
        scat = jnp.where((jnp.broadcast_to(posf, (B, B)) == subf)
                         & (jnp.broadcast_to(kf, (B, B)) > 0.0), 1.0, 0.0)
        shifted = jnp.concatenate(
            [jnp.zeros((B, 1), jnp.float32), cb[:, 0:5], carea,
             jnp.zeros((B, 1), jnp.float32)], axis=1)
        compacted = jnp.dot(scat, shifted, precision=_HI)
        kept_ref[pl.ds(count, B), :] = compacted
        nk = jnp.sum(kf).astype(jnp.int32)
        count = jnp.minimum(count + nk, POST_NMS)
        return b + 1, count

    def outer_cond(state):
        b, count = state
        return (b < NB) & (count < POST_NMS)

    jax.lax.while_loop(outer_cond, outer_body, (jnp.int32(0), jnp.int32(0)))

    blob_ref[...] = kept_ref[0:POST_NMS, 0:5]
    sco_ref[...] = kept_ref[0:POST_NMS, 5:6]


@functools.partial(jax.jit, static_argnames=())
def kernel(boxes, scores, im_info):
    scores_sorted, order = jax.lax.top_k(scores, PRE_NMS)
    props = boxes[order]
    lanes = jnp.zeros((4, NPAD), jnp.float32).at[:, :PRE_NMS].set(props.T)
    lanes3 = lanes.reshape(4, NB, B).transpose(1, 0, 2)
    cols = jnp.zeros((NPAD, 8), jnp.float32)
    cols = cols.at[:PRE_NMS, 0:4].set(props)
    cols = cols.at[:PRE_NMS, 4].set(scores_sorted)
    cols3 = cols.reshape(NB, B, 8)

    blob, out_scores = pl.pallas_call(
        _nms_kernel,
        out_shape=[
            jax.ShapeDtypeStruct((POST_NMS, 5), jnp.float32),
            jax.ShapeDtypeStruct((POST_NMS, 1), jnp.float32),
        ],
        scratch_shapes=[
            pltpu.VMEM((KCAP, 8), jnp.float32),
        ],
    )(lanes3, cols3, im_info.reshape(1, 3))
    return blob, out_scores


# speedup vs baseline: 5.0393x; 1.0729x over previous
import functools

import jax
import jax.numpy as jnp
from jax.experimental import pallas as pl
from jax.experimental.pallas import tpu as pltpu

N_BOXES = 20000
PRE_NMS = 12000
POST_NMS = 2000
THRESH = 0.7

B = 256
NB = (PRE_NMS + B - 1) // B
NPAD = NB * B
KCAP = 2304
CHUNK = 64

_HI = jax.lax.Precision.HIGHEST


def _nms_kernel(lanes_ref, cols_ref, im_ref, blob_ref, sco_ref, kept_ref):
    kept_ref[...] = jnp.zeros((KCAP, 8), jnp.float32)

    w = im_ref[0, 1]
    h = im_ref[0, 0]
    row4 = jax.lax.broadcasted_iota(jnp.int32, (4, 1), 0)
    hi_bound = jnp.where(row4 % 2 == 0, w - 1.0, h - 1.0)
    col8 = jax.lax.broadcasted_iota(jnp.int32, (1, 8), 1)
    colhi = jnp.where(col8 % 2 == 0, w - 1.0, h - 1.0)

    lane = jax.lax.broadcasted_iota(jnp.int32, (1, B), 1)
    row64 = jax.lax.broadcasted_iota(jnp.int32, (CHUNK, 1), 0)
    subi = jax.lax.broadcasted_iota(jnp.int32, (B, B), 0)
    lani = jax.lax.broadcasted_iota(jnp.int32, (B, B), 1)
    trif = jnp.where(subi < lani, 1.0, 0.0).astype(jnp.float32)
    subf = subi.astype(jnp.float32)

    def outer_body(state):
        b, count = state
        blk = lanes_ref[b]
        blk = jnp.minimum(jnp.maximum(blk, 0.0), hi_bound)
        bx1 = blk[0:1, :]
        by1 = blk[1:2, :]
        bx2 = blk[2:3, :]
        by2 = blk[3:4, :]
        barea = (bx2 - bx1 + 1.0) * (by2 - by1 + 1.0)

        cb = cols_ref[b]
        cb = jnp.where(col8 < 4,
                       jnp.minimum(jnp.maximum(cb, 0.0), colhi), cb)
        cx1 = cb[:, 0:1]
        cy1 = cb[:, 1:2]
        cx2 = cb[:, 2:3]
        cy2 = cb[:, 3:4]
        carea = (cx2 - cx1 + 1.0) * (cy2 - cy1 + 1.0)

        mxx1 = jnp.maximum(cx1, bx1)
        myy1 = jnp.maximum(cy1, by1)
        mxx2 = jnp.minimum(cx2, bx2)
        myy2 = jnp.minimum(cy2, by2)
        miw = jnp.maximum(0.0, mxx2 - mxx1 + 1.0)
        mih = jnp.maximum(0.0, myy2 - myy1 + 1.0)
        minter = miw * mih
        miou = minter / (carea + barea - minter)
        tm = jnp.where((miou >= THRESH) & (subi < lani), 1.0, 0.0)

        valid0 = ((b * B + lane) < PRE_NMS).astype(jnp.float32)

        nchunks = (count + CHUNK - 1) // CHUNK

        def chunk_body(c, bval):
            kc = kept_ref[pl.ds(c * CHUNK, CHUNK), :]
            kx1 = kc[:, 1:2]
            ky1 = kc[:, 2:3]
            kx2 = kc[:, 3:4]
            ky2 = kc[:, 4:5]
            karea = kc[:, 6:7]
            xx1 = jnp.maximum(kx1, bx1)
            yy1 = jnp.maximum(ky1, by1)
            xx2 = jnp.minimum(kx2, bx2)
            yy2 = jnp.minimum(ky2, by2)
            iw = jnp.maximum(0.0, xx2 - xx1 + 1.0)
            ih = jnp.maximum(0.0, yy2 - yy1 + 1.0)
            inter = iw * ih
            iou = inter / (karea + barea - inter)
            rowok = (c * CHUNK + row64) < count
            supp = jnp.where((iou >= THRESH) & rowok, 1.0, 0.0)
            supp = jnp.max(supp, axis=0, keepdims=True)
            return jnp.where(supp > 0.0, 0.0, bval)

        bvalid = jax.lax.fori_loop(0, nchunks, chunk_body, valid0)

        def rcond(st):
            kcand, _ = st
            return jnp.max(kcand) > 0.0

        def rbody(st):
            kcand, kf = st
            pred = jnp.dot(kcand, tm)
            dfin = jnp.where(pred > 0.0, 0.0, kcand)
            rsup = jnp.dot(dfin, tm)
            kcand = jnp.where((dfin + rsup) > 0.0, 0.0, kcand)
            return kcand, kf + dfin

        _, kf = jax.lax.while_loop(
            rcond, rbody, (bvalid, jnp.zeros((1, B), jnp.float32)))

        posf = jnp.dot(kf, trif)
        scat = jnp.where((jnp.broadcast_to(posf, (B, B)) == subf)
                         & (jnp.broadcast_to(kf, (B, B)) > 0.0), 1.0, 0.0)
        shifted = jnp.concatenate(
            [jnp.zeros((B, 1), jnp.float32), cb[:, 0:5], carea,
             jnp.zeros((B, 1), jnp.float32)], axis=1)
        compacted = jnp.dot(scat, shifted, precision=_HI)
        kept_ref[pl.ds(count, B), :] = compacted
        nk = jnp.sum(kf).astype(jnp.int32)
        count = jnp.minimum(count + nk, POST_NMS)
        return b + 1, count

    def outer_cond(state):
        b, count = state
        return (b < NB) & (count < POST_NMS)

    jax.lax.while_loop(outer_cond, outer_body, (jnp.int32(0), jnp.int32(0)))

    blob_ref[...] = kept_ref[0:POST_NMS, 0:5]
    sco_ref[...] = kept_ref[0:POST_NMS, 5:6]


@functools.partial(jax.jit, static_argnames=())
def kernel(boxes, scores, im_info):
    scores_sorted, order = jax.lax.top_k(scores, PRE_NMS)
    props = boxes[order]
    lanes = jnp.zeros((4, NPAD), jnp.float32).at[:, :PRE_NMS].set(props.T)
    lanes3 = lanes.reshape(4, NB, B).transpose(1, 0, 2)
    cols = jnp.zeros((NPAD, 8), jnp.float32)
    cols = cols.at[:PRE_NMS, 0:4].set(props)
    cols = cols.at[:PRE_NMS, 4].set(scores_sorted)
    cols3 = cols.reshape(NB, B, 8)

    blob, out_scores = pl.pallas_call(
        _nms_kernel,
        out_shape=[
            jax.ShapeDtypeStruct((POST_NMS, 5), jnp.float32),
            jax.ShapeDtypeStruct((POST_NMS, 1), jnp.float32),
        ],
        scratch_shapes=[
            pltpu.VMEM((KCAP, 8), jnp.float32),
        ],
    )(lanes3, cols3, im_info.reshape(1, 3))
    return blob, out_scores
